# Initial kernel scaffold; baseline (speedup 1.0000x reference)
import jax
import jax.numpy as jnp
from jax.experimental import pallas as pl


def kernel(z_e, codebook):
    raise NotImplementedError("write your pallas kernel here")


# speedup vs baseline: 1.9232x; 1.9232x over previous
import functools

import jax
import jax.numpy as jnp
from jax.experimental import pallas as pl

NUM_CODES = 512
CODE_DIM = 32
BATCH = 65536
TILE = 4096


def _vq_kernel(z_ref, cb_ref, zq_ref, idx_ref, loss_ref):
    i = pl.program_id(0)
    z = z_ref[...]
    cb = cb_ref[...]

    z2 = jnp.sum(z * z, axis=1, keepdims=True)
    cb2 = jnp.sum(cb * cb, axis=1)
    dot = jax.lax.dot_general(
        z, cb,
        dimension_numbers=(((1,), (1,)), ((), ())),
        preferred_element_type=jnp.float32,
    )
    d = z2 - 2.0 * dot + cb2[None, :]
    idx = jnp.argmin(d, axis=1).astype(jnp.int32)
    idx_ref[...] = idx

    onehot = (jax.lax.broadcasted_iota(jnp.int32, (TILE, NUM_CODES), 1)
              == idx[:, None]).astype(jnp.bfloat16)
    cb_hi = cb.astype(jnp.bfloat16)
    cb_lo = (cb - cb_hi.astype(jnp.float32)).astype(jnp.bfloat16)
    dn = (((1,), (0,)), ((), ()))
    zq = (jax.lax.dot_general(onehot, cb_hi, dn,
                              preferred_element_type=jnp.float32)
          + jax.lax.dot_general(onehot, cb_lo, dn,
                                preferred_element_type=jnp.float32))
    zq_ref[...] = zq

    diff = z - zq
    part = jnp.sum(diff * diff)
    acc = jnp.where(i == 0, jnp.zeros((1, 1), jnp.float32), loss_ref[...])
    loss_ref[...] = acc + part


@jax.jit
def kernel(z_e, codebook):
    grid = (BATCH // TILE,)
    zq, idx, loss = pl.pallas_call(
        _vq_kernel,
        grid=grid,
        in_specs=[
            pl.BlockSpec((TILE, CODE_DIM), lambda i: (i, 0)),
            pl.BlockSpec((NUM_CODES, CODE_DIM), lambda i: (0, 0)),
        ],
        out_specs=[
            pl.BlockSpec((TILE, CODE_DIM), lambda i: (i, 0)),
            pl.BlockSpec((TILE,), lambda i: (i,)),
            pl.BlockSpec((1, 1), lambda i: (0, 0)),
        ],
        out_shape=[
            jax.ShapeDtypeStruct((BATCH, CODE_DIM), jnp.float32),
            jax.ShapeDtypeStruct((BATCH,), jnp.int32),
            jax.ShapeDtypeStruct((1, 1), jnp.float32),
        ],
    )(z_e, codebook)
    commitment_loss = loss[0, 0] / (BATCH * CODE_DIM)
    return (zq, idx, commitment_loss)


# speedup vs baseline: 2.6461x; 1.3759x over previous
import functools

import jax
import jax.numpy as jnp
from jax import lax
from jax.experimental import pallas as pl
from jax.experimental.pallas import tpu as pltpu
from jax.experimental.pallas import tpu_sc as plsc

NUM_CODES = 512
CODE_DIM = 32
BATCH = 65536
TILE = 4096


def _argmin_kernel(zt_ref, cb_ref, idx_ref, loss_ref):
    i = pl.program_id(0)
    zt = zt_ref[...]
    cb = cb_ref[...]

    z2 = jnp.sum(zt * zt, axis=0, keepdims=True)
    cb2 = jnp.sum(cb * cb, axis=1, keepdims=True)
    dot = jax.lax.dot_general(
        cb, zt,
        dimension_numbers=(((1,), (0,)), ((), ())),
        preferred_element_type=jnp.float32,
    )
    d = (z2 - 2.0 * dot) + cb2
    m = jnp.min(d, axis=0, keepdims=True)
    code_iota = jax.lax.broadcasted_iota(jnp.int32, (NUM_CODES, TILE), 0)
    idx = jnp.min(jnp.where(d == m, code_iota, NUM_CODES),
                  axis=0, keepdims=True)
    idx_ref[...] = idx.reshape(TILE)

    part = jnp.sum(m)
    acc = jnp.where(i == 0, jnp.zeros((1, 1), jnp.float32), loss_ref[...])
    loss_ref[...] = acc + part


def _tc_argmin(z_e, codebook):
    zt = z_e.T
    idx, loss = pl.pallas_call(
        _argmin_kernel,
        grid=(BATCH // TILE,),
        in_specs=[
            pl.BlockSpec((CODE_DIM, TILE), lambda i: (0, i)),
            pl.BlockSpec((NUM_CODES, CODE_DIM), lambda i: (0, 0)),
        ],
        out_specs=[
            pl.BlockSpec((TILE,), lambda i: (i,)),
            pl.BlockSpec((1, 1), lambda i: (0, 0)),
        ],
        out_shape=[
            jax.ShapeDtypeStruct((BATCH,), jnp.int32),
            jax.ShapeDtypeStruct((1, 1), jnp.float32),
        ],
    )(zt, codebook)
    return idx, loss[0, 0] / (BATCH * CODE_DIM)


_SC_CORES = 2
_SC_SUBCORES = 16


def _sc_gather(codebook, idx):
    nw = _SC_CORES * _SC_SUBCORES
    b_per_w = BATCH // nw
    mesh = plsc.VectorSubcoreMesh(core_axis_name="c", subcore_axis_name="s")

    @functools.partial(
        pl.kernel, mesh=mesh,
        compiler_params=pltpu.CompilerParams(use_tc_tiling_on_sc=False),
        out_type=jax.ShapeDtypeStruct((BATCH, CODE_DIM), jnp.float32),
        scratch_types=[
            pltpu.VMEM((b_per_w,), jnp.int32),
            pltpu.VMEM((b_per_w, CODE_DIM), jnp.float32),
            pltpu.SemaphoreType.DMA,
        ],
    )
    def gather(table_hbm, idx_hbm, out_hbm, idx_v, rows_v, sem):
        wid = lax.axis_index("s") * _SC_CORES + lax.axis_index("c")
        base = wid * b_per_w
        pltpu.sync_copy(idx_hbm.at[pl.ds(base, b_per_w)], idx_v)
        pltpu.async_copy(table_hbm.at[idx_v], rows_v, sem).wait()
        pltpu.sync_copy(rows_v, out_hbm.at[pl.ds(base, b_per_w)])

    return gather(codebook, idx)


@jax.jit
def kernel(z_e, codebook):
    idx, commitment_loss = _tc_argmin(z_e, codebook)
    zq = _sc_gather(codebook, idx)
    return (zq, idx, commitment_loss)
